# baseline (speedup 1.0000x reference)
import functools

import jax
import jax.numpy as jnp
from jax import lax
from jax.experimental import pallas as pl
from jax.experimental.pallas import tpu as pltpu
from jax.experimental.pallas import tpu_sc as plsc

VOCAB = 1000000
D = 32
B = 16384

NC = 2
NS = 16
NW = NC * NS
BPW = B // NW
IROWS = 4
CH = 128
NCH = BPW // CH

BLK = 8192
RPB = BLK // 4
SUB = 4
GL = SUB * BLK
NSTEP = (VOCAB + GL - 1) // GL
RPT = NSTEP * SUB * RPB


def _repack_body(tt_ref, rp_ref):
    x = tt_ref[...]
    ii = lax.broadcasted_iota(jnp.int32, (D, D), 0)
    jj = lax.broadcasted_iota(jnp.int32, (D, D), 1)
    eye = (ii == jj).astype(jnp.float32)
    xt = lax.dot_general(x, eye, (((0,), (0,)), ((), ())),
                         preferred_element_type=jnp.float32)
    rows = []
    for sub in range(SUB):
        parts = [xt[sub * BLK + k * RPB: sub * BLK + (k + 1) * RPB]
                 for k in range(4)]
        rows.append(jnp.concatenate(parts, axis=1))
    rp_ref[...] = jnp.concatenate(rows, axis=0)


def _repack(tt):
    return pl.pallas_call(
        _repack_body,
        grid=(NSTEP,),
        in_specs=[pl.BlockSpec((D, GL), lambda j: (0, j))],
        out_specs=pl.BlockSpec((SUB * RPB, 4 * D), lambda j: (j, 0)),
        out_shape=jax.ShapeDtypeStruct((RPT, 4 * D), jnp.float32),
    )(tt)


def _gather_body(rp0_hbm, rp1_hbm, idx0_hbm, idx1_hbm,
                 pred_hbm, part_hbm,
                 idx_v, q_v, rows_v, out_v, acc_v, sem):
    wid = lax.axis_index("s") * NC + lax.axis_index("c")
    base = wid * BPW
    irow = wid * IROWS

    def do_table(rp_hbm, idx_hbm, out_base, acc):
        pltpu.sync_copy(idx_hbm.at[pl.ds(irow, IROWS)], idx_v)
        for j in range(IROWS):
            for k in range(CH // 16):
                v = idx_v[j, pl.ds(k * 16, 16)]
                q_v[j, pl.ds(k * 16, 16)] = lax.shift_right_logical(v, 2)

        def chunk_body(ch, acc):
            pltpu.async_copy(
                rp_hbm.at[q_v.at[ch]], rows_v, sem).wait()

            def g_body(g, acc):
                r16 = jnp.bitwise_and(idx_v[ch, pl.ds(g * 16, 16)], 3)
                for l in range(16):
                    i = g * 16 + l
                    r = r16[l] * D
                    lo = rows_v[i, pl.ds(r, 16)]
                    hi = rows_v[i, pl.ds(r + 16, 16)]
                    out_v[i, pl.ds(0, 16)] = lo
                    out_v[i, pl.ds(16, 16)] = hi
                    acc = acc + lo + hi
                return acc

            acc = lax.fori_loop(0, CH // 16, g_body, acc)
            pltpu.sync_copy(out_v, pred_hbm.at[pl.ds(out_base + ch * CH, CH)])
            return acc

        return lax.fori_loop(0, NCH, chunk_body, acc)

    acc = do_table(rp0_hbm, idx0_hbm, base, jnp.zeros((16,), jnp.float32))
    acc = do_table(rp1_hbm, idx1_hbm, B + base, acc)
    acc_v[...] = acc
    pltpu.sync_copy(acc_v, part_hbm.at[wid])


@jax.jit
def _sc_lookup(t0, t1, i0, i1):
    rp0 = jnp.reshape(t0, (VOCAB // 4, 4 * D))
    rp1 = jnp.reshape(t1, (VOCAB // 4, 4 * D))

    mesh = plsc.VectorSubcoreMesh(core_axis_name="c", subcore_axis_name="s")
    f = functools.partial(
        pl.kernel, mesh=mesh,
        out_type=[
            jax.ShapeDtypeStruct((2 * B, D), jnp.float32),
            jax.ShapeDtypeStruct((NW, 16), jnp.float32),
        ],
        scratch_types=[
            pltpu.VMEM((IROWS, CH), jnp.int32),
            pltpu.VMEM((IROWS, CH), jnp.int32),
            pltpu.VMEM((CH, 4 * D), jnp.float32),
            pltpu.VMEM((CH, D), jnp.float32),
            pltpu.VMEM((16,), jnp.float32),
            pltpu.SemaphoreType.DMA,
        ],
    )(_gather_body)
    pred, partials = f(rp0, rp1,
                       i0.reshape(NW * IROWS, CH),
                       i1.reshape(NW * IROWS, CH))
    return pred, partials


def kernel(table_0, table_1, indices_0, indices_1):
    pred, partials = _sc_lookup(table_0, table_1, indices_0, indices_1)
    loss = jnp.sum(partials) / jnp.float32(2 * B * D)
    return (loss, pred)


# speedup vs baseline: 1.7233x; 1.7233x over previous
import functools

import jax
import jax.numpy as jnp
from jax import lax
from jax.experimental import pallas as pl
from jax.experimental.pallas import tpu as pltpu
from jax.experimental.pallas import tpu_sc as plsc

VOCAB = 1000000
D = 32
B = 16384

NC = 2
NS = 16
NW = NC * NS
BPW = B // NW
IROWS = 4
CH = 128
NCH = BPW // CH

BLK = 8192
RPB = BLK // 4
SUB = 4
GL = SUB * BLK
NSTEP = (VOCAB + GL - 1) // GL
RPT = NSTEP * SUB * RPB


def _repack_body(tt_ref, rp_ref):
    x = tt_ref[...]
    xt = x.T
    rows = []
    for sub in range(SUB):
        parts = [xt[sub * BLK + k * RPB: sub * BLK + (k + 1) * RPB]
                 for k in range(4)]
        rows.append(jnp.concatenate(parts, axis=1))
    rp_ref[...] = jnp.concatenate(rows, axis=0)


def _repack(tt):
    return pl.pallas_call(
        _repack_body,
        grid=(NSTEP,),
        in_specs=[pl.BlockSpec((D, GL), lambda j: (0, j))],
        out_specs=pl.BlockSpec((SUB * RPB, 4 * D), lambda j: (j, 0)),
        out_shape=jax.ShapeDtypeStruct((RPT, 4 * D), jnp.float32),
    )(tt)


def _gather_body(rp0_hbm, rp1_hbm, idx0_hbm, idx1_hbm,
                 pred_hbm, part_hbm,
                 idx_v, q_v, rows_v, out_v, acc_v, sem):
    wid = lax.axis_index("s") * NC + lax.axis_index("c")
    base = wid * BPW
    irow = wid * IROWS

    def do_table(rp_hbm, idx_hbm, out_base, acc):
        pltpu.sync_copy(idx_hbm.at[pl.ds(irow, IROWS)], idx_v)
        for j in range(IROWS):
            for k in range(CH // 16):
                v = idx_v[j, pl.ds(k * 16, 16)]
                q_v[j, pl.ds(k * 16, 16)] = jnp.bitwise_or(
                    lax.shift_left(lax.shift_right_logical(v, 13), 11),
                    jnp.bitwise_and(v, RPB - 1))

        def chunk_body(ch, acc):
            pltpu.async_copy(
                rp_hbm.at[q_v.at[ch]], rows_v, sem).wait()

            def g_body(g, acc):
                r16 = jnp.bitwise_and(lax.shift_right_logical(
                    idx_v[ch, pl.ds(g * 16, 16)], 11), 3)
                for l in range(16):
                    i = g * 16 + l
                    r = r16[l] * D
                    lo = rows_v[i, pl.ds(r, 16)]
                    hi = rows_v[i, pl.ds(r + 16, 16)]
                    out_v[i, pl.ds(0, 16)] = lo
                    out_v[i, pl.ds(16, 16)] = hi
                    acc = acc + lo + hi
                return acc

            acc = lax.fori_loop(0, CH // 16, g_body, acc)
            pltpu.sync_copy(out_v, pred_hbm.at[pl.ds(out_base + ch * CH, CH)])
            return acc

        return lax.fori_loop(0, NCH, chunk_body, acc)

    acc = do_table(rp0_hbm, idx0_hbm, base, jnp.zeros((16,), jnp.float32))
    acc = do_table(rp1_hbm, idx1_hbm, B + base, acc)
    acc_v[...] = acc
    pltpu.sync_copy(acc_v, part_hbm.at[wid])


@jax.jit
def _sc_lookup(t0, t1, i0, i1):
    rp0 = _repack(t0.T)
    rp1 = _repack(t1.T)

    mesh = plsc.VectorSubcoreMesh(core_axis_name="c", subcore_axis_name="s")
    f = functools.partial(
        pl.kernel, mesh=mesh,
        out_type=[
            jax.ShapeDtypeStruct((2 * B, D), jnp.float32),
            jax.ShapeDtypeStruct((NW, 16), jnp.float32),
        ],
        scratch_types=[
            pltpu.VMEM((IROWS, CH), jnp.int32),
            pltpu.VMEM((IROWS, CH), jnp.int32),
            pltpu.VMEM((CH, 4 * D), jnp.float32),
            pltpu.VMEM((CH, D), jnp.float32),
            pltpu.VMEM((16,), jnp.float32),
            pltpu.SemaphoreType.DMA,
        ],
    )(_gather_body)
    pred, partials = f(rp0, rp1,
                       i0.reshape(NW * IROWS, CH),
                       i1.reshape(NW * IROWS, CH))
    return pred, partials


def kernel(table_0, table_1, indices_0, indices_1):
    pred, partials = _sc_lookup(table_0, table_1, indices_0, indices_1)
    loss = jnp.sum(partials) / jnp.float32(2 * B * D)
    return (loss, pred)
